# baseline (speedup 1.0000x reference)
import functools

import jax
import jax.numpy as jnp
from jax import lax
from jax.experimental import pallas as pl
from jax.experimental.pallas import tpu as pltpu
from jax.experimental.pallas import tpu_sc as plsc

BATCH = 16384
EMB = 32
NROW = 1000000
TWB = 17
TW = 1 << TWB
W8B = TWB - 3
W8 = TW // 8
NBLK = (NROW + TW - 1) // TW
NPACK = NBLK * W8
NC = 2
NS = 16
NW = NC * NS
BPW = BATCH // NW
HPW = BPW // 2
GRP = 16


def _relayout_body(x_ref, o_ref):
    x = x_ref[...]
    a = jnp.concatenate([x[:, k * W8:(k + 1) * W8] for k in range(4)], axis=0)
    b = jnp.concatenate([x[:, k * W8:(k + 1) * W8] for k in range(4, 8)], axis=0)
    ua = lax.bitcast_convert_type(
        lax.convert_element_type(a, jnp.bfloat16).T, jnp.uint16)
    ub = lax.bitcast_convert_type(
        lax.convert_element_type(b, jnp.bfloat16).T, jnp.uint16)
    packed = (ua.astype(jnp.uint32) << 16) | ub.astype(jnp.uint32)
    o_ref[...] = lax.bitcast_convert_type(packed, jnp.int32)


def _relayout(tbl_t):
    return pl.pallas_call(
        _relayout_body,
        grid=(pl.cdiv(NROW, TW),),
        in_specs=[pl.BlockSpec((EMB, TW), lambda i: (0, i))],
        out_specs=pl.BlockSpec((W8, 128), lambda i: (i, 0)),
        out_shape=jax.ShapeDtypeStruct((NPACK, 128), jnp.int32),
    )(tbl_t)


def _bits_to_f32(g, is_hi):
    hi = plsc.bitcast((g >> 16) << 16, jnp.float32)
    lo = plsc.bitcast(g << 16, jnp.float32)
    return jnp.where(is_hi, hi, lo)


def _mf_kernel(user_hbm, item_hbm, upk_hbm, ipk_hbm, out_hbm,
               uids_v, iids_v, uq_v, iq_v, ubuf_v, vbuf_v, out_v,
               sem_u0, sem_u1, sem_i0, sem_i1):
    sems_u = (sem_u0, sem_u1)
    sems_i = (sem_i0, sem_i1)
    wid = lax.axis_index("s") * NC + lax.axis_index("c")
    base = wid * BPW

    pltpu.sync_copy(user_hbm.at[pl.ds(base, BPW)], uids_v)
    pltpu.sync_copy(item_hbm.at[pl.ds(base, BPW)], iids_v)

    lanes = lax.iota(jnp.int32, GRP)

    for p in range(2):
        poff = p * HPW

        def qbody(g, carry):
            off = pl.multiple_of(g * GRP, GRP)
            rv = uids_v[pl.ds(poff + off, GRP)]
            sv = iids_v[pl.ds(poff + off, GRP)]
            uq_v[pl.ds(off, GRP)] = ((rv >> TWB) << W8B) | (rv & (W8 - 1))
            iq_v[pl.ds(off, GRP)] = ((sv >> TWB) << W8B) | (sv & (W8 - 1))
            return carry
        lax.fori_loop(0, HPW // GRP, qbody, 0)

        for j in range(HPW // 128):
            pltpu.async_copy(upk_hbm.at[uq_v.at[pl.ds(j * 128, 128)]],
                             ubuf_v.at[pl.ds(j * 128, 128)], sems_u[j])
            pltpu.async_copy(ipk_hbm.at[iq_v.at[pl.ds(j * 128, 128)]],
                             vbuf_v.at[pl.ds(j * 128, 128)], sems_i[j])
        def grp_body(g, carry):
            off = pl.multiple_of(g * GRP, GRP)
            rv = uids_v[pl.ds(poff + off, GRP)]
            sv = iids_v[pl.ds(poff + off, GRP)]
            uk = (rv >> W8B) & 7
            ik = (sv >> W8B) & 7
            usub = (uk & 3) << 5
            isub = (ik & 3) << 5
            u_hi = uk < 4
            i_hi = ik < 4
            rows = off + lanes
            acc = jnp.zeros((GRP,), jnp.float32)
            for c in range(EMB):
                gu = plsc.load_gather(ubuf_v, [rows, usub + c])
                gi = plsc.load_gather(vbuf_v, [rows, isub + c])
                acc = acc + _bits_to_f32(gu, u_hi) * _bits_to_f32(gi, i_hi)
            out_v[pl.ds(off, GRP)] = acc
            return carry

        gpc = 128 // GRP
        for j in range(HPW // 128):
            pltpu.make_async_copy(upk_hbm.at[pl.ds(0, 128)],
                                  ubuf_v.at[pl.ds(j * 128, 128)], sems_u[j]).wait()
            pltpu.make_async_copy(ipk_hbm.at[pl.ds(0, 128)],
                                  vbuf_v.at[pl.ds(j * 128, 128)], sems_i[j]).wait()
            lax.fori_loop(j * gpc, (j + 1) * gpc, grp_body, 0)

        pltpu.sync_copy(out_v, out_hbm.at[pl.ds(base + poff, HPW)])


def kernel(user, item, user_emb, item_emb):
    upk = _relayout(user_emb.T)
    ipk = _relayout(item_emb.T)
    k = functools.partial(
        pl.kernel,
        mesh=plsc.VectorSubcoreMesh(core_axis_name="c", subcore_axis_name="s"),
        compiler_params=pltpu.CompilerParams(
            needs_layout_passes=False, use_tc_tiling_on_sc=True),
        out_type=jax.ShapeDtypeStruct((BATCH,), jnp.float32),
        scratch_types=[
            pltpu.VMEM((BPW,), jnp.int32),
            pltpu.VMEM((BPW,), jnp.int32),
            pltpu.VMEM((HPW,), jnp.int32),
            pltpu.VMEM((HPW,), jnp.int32),
            pltpu.VMEM((HPW, 128), jnp.int32),
            pltpu.VMEM((HPW, 128), jnp.int32),
            pltpu.VMEM((HPW,), jnp.float32),
            pltpu.SemaphoreType.DMA,
            pltpu.SemaphoreType.DMA,
            pltpu.SemaphoreType.DMA,
            pltpu.SemaphoreType.DMA,
        ],
    )(_mf_kernel)
    return k(user.astype(jnp.int32), item.astype(jnp.int32), upk, ipk)
